# Initial kernel scaffold; baseline (speedup 1.0000x reference)
import jax
import jax.numpy as jnp
from jax.experimental import pallas as pl


def kernel(feature, sample_offsets, batch_index):
    raise NotImplementedError("write your pallas kernel here")


# speedup vs baseline: 2.0349x; 2.0349x over previous
import functools

import jax
import jax.numpy as jnp
from jax import lax
from jax.experimental import pallas as pl
from jax.experimental.pallas import tpu as pltpu
from jax.experimental.pallas import tpu_sc as plsc

B = 16
L = 2048
N = 16384
D = 512

NS = 16
ROWS_PER_SC = (B * L) // NS
IN_PER_SC = N // NS
ZROWS = 64
CH = 64
CHUNKS = IN_PER_SC // CH


def _build():
    mesh = plsc.VectorSubcoreMesh(
        core_axis_name="c", subcore_axis_name="s", num_cores=1
    )

    @functools.partial(
        pl.kernel,
        out_type=jax.ShapeDtypeStruct((B * L, D), jnp.float32),
        mesh=mesh,
        scratch_types=[
            pltpu.VMEM((ZROWS, D), jnp.float32),
            pltpu.VMEM((CH, D), jnp.float32),
            pltpu.VMEM((CHUNKS, CH), jnp.int32),
            pltpu.VMEM((CHUNKS, CH), jnp.int32),
            pltpu.SemaphoreType.DMA,
        ],
    )
    def scatter_kernel(feat_hbm, off_hbm, bidx_hbm, z_hbm, out_hbm,
                       zbuf, rbuf, dstv, bv, zsem):
        sid = lax.axis_index("s")

        pltpu.sync_copy(z_hbm, zbuf)

        zbase = sid * ROWS_PER_SC
        zero_copies = []
        for i in range(ROWS_PER_SC // ZROWS):
            zero_copies.append(
                pltpu.async_copy(
                    zbuf, out_hbm.at[pl.ds(zbase + i * ZROWS, ZROWS)], zsem
                )
            )

        pltpu.sync_copy(off_hbm.at[sid], dstv)
        pltpu.sync_copy(bidx_hbm.at[sid], bv)

        @pl.loop(0, CHUNKS)
        def _(ch):
            @pl.loop(0, CH, step=16)
            def _(j):
                s = pl.ds(j, 16)
                dstv[ch, s] = bv[ch, s] * L + dstv[ch, s]

        for c in zero_copies:
            c.wait()
        plsc.subcore_barrier()

        ibase = sid * IN_PER_SC
        for ch in range(CHUNKS):
            pltpu.sync_copy(feat_hbm.at[pl.ds(ibase + ch * CH, CH)], rbuf)
            pltpu.sync_copy(rbuf, out_hbm.at[dstv.at[ch]])

    return scatter_kernel


_scatter = _build()


@jax.jit
def _run(feature, sample_offsets, batch_index):
    zblk = jnp.zeros((ZROWS, D), jnp.float32)
    off3 = sample_offsets.reshape(NS, CHUNKS, CH)
    bidx3 = batch_index.reshape(NS, CHUNKS, CH)
    out = _scatter(feature, off3, bidx3, zblk)
    return out.reshape(B, L, D)


def kernel(feature, sample_offsets, batch_index):
    return _run(feature, sample_offsets, batch_index)


# speedup vs baseline: 3.3562x; 1.6493x over previous
import jax
import jax.numpy as jnp
from jax import lax
from jax.experimental import pallas as pl
from jax.experimental.pallas import tpu as pltpu
from jax.experimental.pallas import tpu_sc as plsc

B = 16
L = 2048
N = 16384
D = 512

NC = 2
NS = 16
NW = NC * NS
IN_PER_W = N // NW
CH = 64
CHUNKS = IN_PER_W // CH

_mesh = plsc.VectorSubcoreMesh(
    core_axis_name="c", subcore_axis_name="s", num_cores=NC
)


@jax.jit
def _run(feature, sample_offsets, batch_index):
    off3 = sample_offsets.reshape(NW, CHUNKS, CH)
    bidx3 = batch_index.reshape(NW, CHUNKS, CH)
    out0 = jnp.zeros((B * L, D), jnp.float32)

    def stateful(refs):
        feat_hbm, off_hbm, bidx_hbm, out_hbm = refs

        @pl.core_map(
            _mesh,
            scratch_shapes=[
                pltpu.VMEM((CH, D), jnp.float32),
                pltpu.VMEM((CH, D), jnp.float32),
                pltpu.VMEM((CHUNKS, CH), jnp.int32),
                pltpu.VMEM((CHUNKS, CH), jnp.int32),
                pltpu.SemaphoreType.DMA,
                pltpu.SemaphoreType.DMA,
                pltpu.SemaphoreType.DMA,
                pltpu.SemaphoreType.DMA,
            ],
        )
        def _(rbuf0, rbuf1, dstv, bv, gsem0, gsem1, ssem0, ssem1):
            wid = lax.axis_index("c") * NS + lax.axis_index("s")

            pltpu.sync_copy(off_hbm.at[wid], dstv)
            pltpu.sync_copy(bidx_hbm.at[wid], bv)

            @pl.loop(0, CHUNKS)
            def _(ch):
                @pl.loop(0, CH, step=16)
                def _(j):
                    s = pl.ds(j, 16)
                    dstv[ch, s] = bv[ch, s] * L + dstv[ch, s]

            ibase = wid * IN_PER_W
            rbufs = (rbuf0, rbuf1)
            gsems = (gsem0, gsem1)
            ssems = (ssem0, ssem1)

            gathers = [None, None]
            scatters = [None, None]
            gathers[0] = pltpu.async_copy(
                feat_hbm.at[pl.ds(ibase, CH)], rbufs[0], gsems[0]
            )
            for ch in range(CHUNKS):
                p = ch % 2
                q = (ch + 1) % 2
                if ch + 1 < CHUNKS:
                    if scatters[q] is not None:
                        scatters[q].wait()
                    gathers[q] = pltpu.async_copy(
                        feat_hbm.at[pl.ds(ibase + (ch + 1) * CH, CH)],
                        rbufs[q],
                        gsems[q],
                    )
                gathers[p].wait()
                scatters[p] = pltpu.async_copy(
                    rbufs[p], out_hbm.at[dstv.at[ch]], ssems[p]
                )
            scatters[0].wait()
            scatters[1].wait()

    _, _, _, out = pl.run_state(stateful)((feature, off3, bidx3, out0))
    return out.reshape(B, L, D)


def kernel(feature, sample_offsets, batch_index):
    return _run(feature, sample_offsets, batch_index)


# speedup vs baseline: 3.5637x; 1.0618x over previous
import jax
import jax.numpy as jnp
from jax import lax
from jax.experimental import pallas as pl
from jax.experimental.pallas import tpu as pltpu
from jax.experimental.pallas import tpu_sc as plsc

B = 16
L = 2048
N = 16384
D = 512

NC = 2
NS = 16
NW = NC * NS
IN_PER_W = N // NW
CH = 64
CHUNKS = IN_PER_W // CH
NBUF = 3

_mesh = plsc.VectorSubcoreMesh(
    core_axis_name="c", subcore_axis_name="s", num_cores=NC
)


@jax.jit
def _run(feature, sample_offsets, batch_index):
    out0 = jnp.zeros((B * L, D), jnp.float32)

    def stateful(refs):
        feat_hbm, off_hbm, bidx_hbm, out_hbm = refs

        @pl.core_map(
            _mesh,
            scratch_shapes=[
                [pltpu.VMEM((CH, D), jnp.float32) for _ in range(NBUF)],
                pltpu.VMEM((IN_PER_W,), jnp.int32),
                pltpu.VMEM((IN_PER_W,), jnp.int32),
                pltpu.VMEM((CHUNKS, CH), jnp.int32),
                [pltpu.SemaphoreType.DMA for _ in range(NBUF)],
                [pltpu.SemaphoreType.DMA for _ in range(NBUF)],
            ],
        )
        def _(rbufs, offf, bvf, dstv, gsems, ssems):
            wid = lax.axis_index("c") * NS + lax.axis_index("s")
            ibase = wid * IN_PER_W

            gathers = [None] * CHUNKS
            for ch in range(min(NBUF - 1, CHUNKS)):
                gathers[ch] = pltpu.async_copy(
                    feat_hbm.at[pl.ds(ibase + ch * CH, CH)],
                    rbufs[ch % NBUF],
                    gsems[ch % NBUF],
                )

            pltpu.sync_copy(off_hbm.at[pl.ds(ibase, IN_PER_W)], offf)
            pltpu.sync_copy(bidx_hbm.at[pl.ds(ibase, IN_PER_W)], bvf)

            for ch in range(CHUNKS):
                @pl.loop(0, CH, step=16)
                def _(j, ch=ch):
                    s = pl.ds(ch * CH + j, 16)
                    dstv[ch, pl.ds(j, 16)] = bvf[s] * L + offf[s]

            scatters = [None] * CHUNKS
            for ch in range(CHUNKS):
                gathers[ch].wait()
                scatters[ch] = pltpu.async_copy(
                    rbufs[ch % NBUF], out_hbm.at[dstv.at[ch]], ssems[ch % NBUF]
                )
                nx = ch + NBUF - 1
                if nx < CHUNKS:
                    if scatters[nx - NBUF] is not None:
                        scatters[nx - NBUF].wait()
                    gathers[nx] = pltpu.async_copy(
                        feat_hbm.at[pl.ds(ibase + nx * CH, CH)],
                        rbufs[nx % NBUF],
                        gsems[nx % NBUF],
                    )
            for ch in range(CHUNKS):
                if ch + NBUF >= CHUNKS:
                    scatters[ch].wait()

    _, _, _, out = pl.run_state(
        stateful)((feature, sample_offsets, batch_index, out0))
    return out.reshape(B, L, D)


def kernel(feature, sample_offsets, batch_index):
    return _run(feature, sample_offsets, batch_index)


# speedup vs baseline: 3.6247x; 1.0171x over previous
import jax
import jax.numpy as jnp
from jax import lax
from jax.experimental import pallas as pl
from jax.experimental.pallas import tpu as pltpu
from jax.experimental.pallas import tpu_sc as plsc

B = 16
L = 2048
N = 16384
D = 512

NC = 2
NS = 16
NW = NC * NS
IN_PER_W = N // NW
CH = 32
CHUNKS = IN_PER_W // CH
NBUF = 7

_mesh = plsc.VectorSubcoreMesh(
    core_axis_name="c", subcore_axis_name="s", num_cores=NC
)


@jax.jit
def _run(feature, sample_offsets, batch_index):
    out0 = jnp.zeros((B * L, D), jnp.float32)

    def stateful(refs):
        feat_hbm, off_hbm, bidx_hbm, out_hbm = refs

        @pl.core_map(
            _mesh,
            scratch_shapes=[
                [pltpu.VMEM((CH, D), jnp.float32) for _ in range(NBUF)],
                pltpu.VMEM((IN_PER_W,), jnp.int32),
                pltpu.VMEM((IN_PER_W,), jnp.int32),
                pltpu.VMEM((CHUNKS, CH), jnp.int32),
                [pltpu.SemaphoreType.DMA for _ in range(NBUF)],
                [pltpu.SemaphoreType.DMA for _ in range(NBUF)],
            ],
        )
        def _(rbufs, offf, bvf, dstv, gsems, ssems):
            wid = lax.axis_index("c") * NS + lax.axis_index("s")
            ibase = wid * IN_PER_W

            gathers = [None] * CHUNKS
            for ch in range(min(NBUF - 1, CHUNKS)):
                gathers[ch] = pltpu.async_copy(
                    feat_hbm.at[pl.ds(ibase + ch * CH, CH)],
                    rbufs[ch % NBUF],
                    gsems[ch % NBUF],
                )

            pltpu.sync_copy(off_hbm.at[pl.ds(ibase, IN_PER_W)], offf)
            pltpu.sync_copy(bidx_hbm.at[pl.ds(ibase, IN_PER_W)], bvf)

            for ch in range(CHUNKS):
                @pl.loop(0, CH, step=16)
                def _(j, ch=ch):
                    s = pl.ds(ch * CH + j, 16)
                    dstv[ch, pl.ds(j, 16)] = bvf[s] * L + offf[s]

            scatters = [None] * CHUNKS
            for ch in range(CHUNKS):
                gathers[ch].wait()
                scatters[ch] = pltpu.async_copy(
                    rbufs[ch % NBUF], out_hbm.at[dstv.at[ch]], ssems[ch % NBUF]
                )
                nx = ch + NBUF - 1
                if nx < CHUNKS:
                    if scatters[nx - NBUF] is not None:
                        scatters[nx - NBUF].wait()
                    gathers[nx] = pltpu.async_copy(
                        feat_hbm.at[pl.ds(ibase + nx * CH, CH)],
                        rbufs[nx % NBUF],
                        gsems[nx % NBUF],
                    )
            for ch in range(CHUNKS):
                if ch + NBUF >= CHUNKS:
                    scatters[ch].wait()

    _, _, _, out = pl.run_state(
        stateful)((feature, sample_offsets, batch_index, out0))
    return out.reshape(B, L, D)


def kernel(feature, sample_offsets, batch_index):
    return _run(feature, sample_offsets, batch_index)
